# Initial kernel scaffold; baseline (speedup 1.0000x reference)
import jax
import jax.numpy as jnp
from jax.experimental import pallas as pl


def kernel(sampled_lidar_list, surface_points):
    raise NotImplementedError("write your pallas kernel here")


# speedup vs baseline: 1.1095x; 1.1095x over previous
import functools

import jax
import jax.numpy as jnp
from jax import lax
from jax.experimental import pallas as pl
from jax.experimental.pallas import tpu as pltpu
from jax.experimental.pallas import tpu_sc as plsc

_B = 8
_N = 8192
_M = 4096
_SH = 4
_RS = _N // _SH
_L = 16
_U = 8

_mesh = plsc.VectorSubcoreMesh(core_axis_name="c", subcore_axis_name="s")


@functools.partial(
    pl.kernel,
    mesh=_mesh,
    out_type=[
        jax.ShapeDtypeStruct((_B, _N), jnp.float32),
        jax.ShapeDtypeStruct((_B, _SH, _M), jnp.float32),
    ],
    scratch_types=[
        pltpu.VMEM((_RS,), jnp.float32),
        pltpu.VMEM((_RS,), jnp.float32),
        pltpu.VMEM((_RS,), jnp.float32),
        pltpu.VMEM((_M,), jnp.float32),
        pltpu.VMEM((_M,), jnp.float32),
        pltpu.VMEM((_M,), jnp.float32),
        pltpu.VMEM((_RS,), jnp.float32),
        pltpu.VMEM((_M,), jnp.float32),
    ],
)
def _sc_min(ax_h, ay_h, az_h, bx_h, by_h, bz_h, row_o, col_o,
            axv, ayv, azv, bxv, byv, bzv, rminv, cminv):
    wid = lax.axis_index("s") * 2 + lax.axis_index("c")
    box = wid // _SH
    sh = wid % _SH
    r0 = sh * _RS

    pltpu.sync_copy(ax_h.at[box, pl.ds(r0, _RS)], axv)
    pltpu.sync_copy(ay_h.at[box, pl.ds(r0, _RS)], ayv)
    pltpu.sync_copy(az_h.at[box, pl.ds(r0, _RS)], azv)
    pltpu.sync_copy(bx_h.at[box], bxv)
    pltpu.sync_copy(by_h.at[box], byv)
    pltpu.sync_copy(bz_h.at[box], bzv)

    inf16 = jnp.full((_L,), jnp.inf, jnp.float32)

    def rowblk(i, c):
        a_x = axv[pl.ds(i * _L, _L)]
        a_y = ayv[pl.ds(i * _L, _L)]
        a_z = azv[pl.ds(i * _L, _L)]

        def mstep(m, acc):
            bxu = bxv[pl.ds(m * _L, _L)]
            byu = byv[pl.ds(m * _L, _L)]
            bzu = bzv[pl.ds(m * _L, _L)]
            for u in range(_L):
                dx = a_x - bxu[u]
                dy = a_y - byu[u]
                dz = a_z - bzu[u]
                acc = jnp.minimum(acc, dx * dx + dy * dy + dz * dz)
            return acc

        rminv[pl.ds(i * _L, _L)] = lax.fori_loop(0, _M // _L, mstep, inf16)
        return c

    lax.fori_loop(0, _RS // _L, rowblk, 0)

    def colblk(j, c):
        b_x = bxv[pl.ds(j * _L, _L)]
        b_y = byv[pl.ds(j * _L, _L)]
        b_z = bzv[pl.ds(j * _L, _L)]

        def nstep(n, acc):
            axu = axv[pl.ds(n * _L, _L)]
            ayu = ayv[pl.ds(n * _L, _L)]
            azu = azv[pl.ds(n * _L, _L)]
            for u in range(_L):
                dx = b_x - axu[u]
                dy = b_y - ayu[u]
                dz = b_z - azu[u]
                acc = jnp.minimum(acc, dx * dx + dy * dy + dz * dz)
            return acc

        cminv[pl.ds(j * _L, _L)] = lax.fori_loop(0, _RS // _L, nstep, inf16)
        return c

    lax.fori_loop(0, _M // _L, colblk, 0)

    pltpu.sync_copy(rminv, row_o.at[box, pl.ds(r0, _RS)])
    pltpu.sync_copy(cminv, col_o.at[box, sh])


def _finish_body(rowmin_ref, colminp_ref, out_ref):
    rm2 = rowmin_ref[...]
    cm2 = jnp.min(colminp_ref[...], axis=1)
    rm = jnp.sqrt(jnp.maximum(rm2, 1e-12))
    cm = jnp.sqrt(jnp.maximum(cm2, 1e-12))
    lb = jnp.mean(cm, axis=1)
    lf = jnp.mean(rm, axis=1)
    lm = jnp.max(rm, axis=1)
    out_ref[...] = jnp.mean(5.0 * lb + lf + lm).reshape(1, 1)


_finish = pl.pallas_call(
    _finish_body,
    out_shape=jax.ShapeDtypeStruct((1, 1), jnp.float32),
)


@jax.jit
def kernel(sampled_lidar_list, surface_points):
    a = sampled_lidar_list
    b = surface_points
    ax, ay, az = (a[:, :, 0], a[:, :, 1], a[:, :, 2])
    bx, by, bz = (b[:, :, 0], b[:, :, 1], b[:, :, 2])
    rowmin2, colmin2p = _sc_min(ax, ay, az, bx, by, bz)
    return _finish(rowmin2, colmin2p)[0, 0]


# speedup vs baseline: 3.5323x; 3.1836x over previous
import functools

import jax
import jax.numpy as jnp
from jax import lax
from jax.experimental import pallas as pl
from jax.experimental.pallas import tpu as pltpu
from jax.experimental.pallas import tpu_sc as plsc

_B = 8
_N = 8192
_M = 4096
_L = 16

_NSC = 2048
_SH = 4
_RS = _NSC // _SH

_RB = 512
_NTC = _N - _NSC
_NRB = _NTC // _RB

_mesh = plsc.VectorSubcoreMesh(core_axis_name="c", subcore_axis_name="s")


@functools.partial(
    pl.kernel,
    mesh=_mesh,
    out_type=[
        jax.ShapeDtypeStruct((_B, _NSC), jnp.float32),
        jax.ShapeDtypeStruct((_B, _SH, _M), jnp.float32),
    ],
    scratch_types=[
        pltpu.VMEM((_RS,), jnp.float32),
        pltpu.VMEM((_RS,), jnp.float32),
        pltpu.VMEM((_RS,), jnp.float32),
        pltpu.VMEM((_M,), jnp.float32),
        pltpu.VMEM((_M,), jnp.float32),
        pltpu.VMEM((_M,), jnp.float32),
        pltpu.VMEM((_RS,), jnp.float32),
        pltpu.VMEM((_M,), jnp.float32),
    ],
)
def _sc_min(ax_h, ay_h, az_h, bx_h, by_h, bz_h, row_o, col_o,
            axv, ayv, azv, bxv, byv, bzv, rminv, cminv):
    wid = lax.axis_index("s") * 2 + lax.axis_index("c")
    box = wid // _SH
    sh = wid % _SH
    r0 = sh * _RS

    pltpu.sync_copy(ax_h.at[box, pl.ds(r0, _RS)], axv)
    pltpu.sync_copy(ay_h.at[box, pl.ds(r0, _RS)], ayv)
    pltpu.sync_copy(az_h.at[box, pl.ds(r0, _RS)], azv)
    pltpu.sync_copy(bx_h.at[box], bxv)
    pltpu.sync_copy(by_h.at[box], byv)
    pltpu.sync_copy(bz_h.at[box], bzv)

    inf16 = jnp.full((_L,), jnp.inf, jnp.float32)

    def rowblk(i, c):
        a_x = axv[pl.ds(i * _L, _L)]
        a_y = ayv[pl.ds(i * _L, _L)]
        a_z = azv[pl.ds(i * _L, _L)]

        def mstep(m, acc):
            bxu = bxv[pl.ds(m * _L, _L)]
            byu = byv[pl.ds(m * _L, _L)]
            bzu = bzv[pl.ds(m * _L, _L)]
            for u in range(_L):
                dx = a_x - bxu[u]
                dy = a_y - byu[u]
                dz = a_z - bzu[u]
                acc = jnp.minimum(acc, dx * dx + dy * dy + dz * dz)
            return acc

        rminv[pl.ds(i * _L, _L)] = lax.fori_loop(0, _M // _L, mstep, inf16)
        return c

    lax.fori_loop(0, _RS // _L, rowblk, 0)

    def colblk(j, c):
        b_x = bxv[pl.ds(j * _L, _L)]
        b_y = byv[pl.ds(j * _L, _L)]
        b_z = bzv[pl.ds(j * _L, _L)]

        def nstep(n, acc):
            axu = axv[pl.ds(n * _L, _L)]
            ayu = ayv[pl.ds(n * _L, _L)]
            azu = azv[pl.ds(n * _L, _L)]
            for u in range(_L):
                dx = b_x - axu[u]
                dy = b_y - ayu[u]
                dz = b_z - azu[u]
                acc = jnp.minimum(acc, dx * dx + dy * dy + dz * dz)
            return acc

        cminv[pl.ds(j * _L, _L)] = lax.fori_loop(0, _RS // _L, nstep, inf16)
        return c

    lax.fori_loop(0, _M // _L, colblk, 0)

    pltpu.sync_copy(rminv, row_o.at[box, pl.ds(r0, _RS)])
    pltpu.sync_copy(cminv, col_o.at[box, sh])


def _tc_min_body(a_ref, b_ref, rowmin_ref, colminp_ref):
    a2 = a_ref[0]
    b2 = b_ref[0]
    ip = lax.dot_general(a2, b2, (((1,), (1,)), ((), ())),
                         precision=lax.Precision.HIGHEST)
    na = jnp.sum(a2 * a2, axis=1)
    nb = jnp.sum(b2 * b2, axis=1)
    d2 = (na[:, None] - 2.0 * ip) + nb[None, :]
    rowmin_ref[0, 0, 0] = jnp.min(d2, axis=1)
    colminp_ref[0, 0, 0] = jnp.min(d2, axis=0)


_tc_min = pl.pallas_call(
    _tc_min_body,
    grid=(_B, _NRB),
    in_specs=[
        pl.BlockSpec((1, _RB, 3), lambda b, r: (b, (_NSC // _RB) + r, 0)),
        pl.BlockSpec((1, _M, 3), lambda b, r: (b, 0, 0)),
    ],
    out_specs=[
        pl.BlockSpec((1, 1, 1, _RB), lambda b, r: (b, r, 0, 0)),
        pl.BlockSpec((1, 1, 1, _M), lambda b, r: (b, r, 0, 0)),
    ],
    out_shape=[
        jax.ShapeDtypeStruct((_B, _NRB, 1, _RB), jnp.float32),
        jax.ShapeDtypeStruct((_B, _NRB, 1, _M), jnp.float32),
    ],
)


def _finish_body(rs_ref, rt_ref, cs_ref, ct_ref, out_ref):
    rt2 = rt_ref[...].reshape(_B, _NTC)
    rm2 = jnp.concatenate([rs_ref[...], rt2], axis=1)
    cm2 = jnp.minimum(jnp.min(cs_ref[...], axis=1),
                      jnp.min(ct_ref[...].reshape(_B, _NRB, _M), axis=1))
    rm = jnp.sqrt(jnp.maximum(rm2, 1e-12))
    cm = jnp.sqrt(jnp.maximum(cm2, 1e-12))
    lb = jnp.mean(cm, axis=1)
    lf = jnp.mean(rm, axis=1)
    lm = jnp.max(rm, axis=1)
    out_ref[...] = jnp.mean(5.0 * lb + lf + lm).reshape(1, 1)


_finish = pl.pallas_call(
    _finish_body,
    out_shape=jax.ShapeDtypeStruct((1, 1), jnp.float32),
)


@jax.jit
def kernel(sampled_lidar_list, surface_points):
    a = sampled_lidar_list
    b = surface_points
    ax, ay, az = (a[:, :, 0], a[:, :, 1], a[:, :, 2])
    bx, by, bz = (b[:, :, 0], b[:, :, 1], b[:, :, 2])
    rowmin_sc, colmin_sc = _sc_min(ax, ay, az, bx, by, bz)
    rowmin_tc, colmin_tc = _tc_min(a, b)
    return _finish(rowmin_sc, rowmin_tc, colmin_sc, colmin_tc)[0, 0]


# speedup vs baseline: 5.8803x; 1.6647x over previous
import functools

import jax
import jax.numpy as jnp
from jax import lax
from jax.experimental import pallas as pl
from jax.experimental.pallas import tpu as pltpu
from jax.experimental.pallas import tpu_sc as plsc

_B = 8
_N = 8192
_M = 4096
_L = 16

_NSC = 1024
_SH = 4
_RS = _NSC // _SH

_RB = 512
_NTC = _N - _NSC
_NRB = _NTC // _RB

_mesh = plsc.VectorSubcoreMesh(core_axis_name="c", subcore_axis_name="s")


@functools.partial(
    pl.kernel,
    mesh=_mesh,
    out_type=[
        jax.ShapeDtypeStruct((_B, _NSC), jnp.float32),
        jax.ShapeDtypeStruct((_B, _SH, _M), jnp.float32),
    ],
    scratch_types=[
        pltpu.VMEM((_RS,), jnp.float32),
        pltpu.VMEM((_RS,), jnp.float32),
        pltpu.VMEM((_RS,), jnp.float32),
        pltpu.VMEM((_M,), jnp.float32),
        pltpu.VMEM((_M,), jnp.float32),
        pltpu.VMEM((_M,), jnp.float32),
        pltpu.VMEM((_RS,), jnp.float32),
        pltpu.VMEM((_M,), jnp.float32),
    ],
)
def _sc_min(ax_h, ay_h, az_h, bx_h, by_h, bz_h, row_o, col_o,
            axv, ayv, azv, bxv, byv, bzv, rminv, cminv):
    wid = lax.axis_index("s") * 2 + lax.axis_index("c")
    box = wid // _SH
    sh = wid % _SH
    r0 = sh * _RS

    pltpu.sync_copy(ax_h.at[box, pl.ds(r0, _RS)], axv)
    pltpu.sync_copy(ay_h.at[box, pl.ds(r0, _RS)], ayv)
    pltpu.sync_copy(az_h.at[box, pl.ds(r0, _RS)], azv)
    pltpu.sync_copy(bx_h.at[box], bxv)
    pltpu.sync_copy(by_h.at[box], byv)
    pltpu.sync_copy(bz_h.at[box], bzv)

    inf16 = jnp.full((_L,), jnp.inf, jnp.float32)

    def rowblk(i, c):
        a_x = axv[pl.ds(i * _L, _L)]
        a_y = ayv[pl.ds(i * _L, _L)]
        a_z = azv[pl.ds(i * _L, _L)]

        def mstep(m, acc):
            bxu = bxv[pl.ds(m * _L, _L)]
            byu = byv[pl.ds(m * _L, _L)]
            bzu = bzv[pl.ds(m * _L, _L)]
            for u in range(_L):
                dx = a_x - bxu[u]
                dy = a_y - byu[u]
                dz = a_z - bzu[u]
                acc = jnp.minimum(acc, dx * dx + dy * dy + dz * dz)
            return acc

        rminv[pl.ds(i * _L, _L)] = lax.fori_loop(0, _M // _L, mstep, inf16)
        return c

    lax.fori_loop(0, _RS // _L, rowblk, 0)

    def colblk(j, c):
        b_x = bxv[pl.ds(j * _L, _L)]
        b_y = byv[pl.ds(j * _L, _L)]
        b_z = bzv[pl.ds(j * _L, _L)]

        def nstep(n, acc):
            axu = axv[pl.ds(n * _L, _L)]
            ayu = ayv[pl.ds(n * _L, _L)]
            azu = azv[pl.ds(n * _L, _L)]
            for u in range(_L):
                dx = b_x - axu[u]
                dy = b_y - ayu[u]
                dz = b_z - azu[u]
                acc = jnp.minimum(acc, dx * dx + dy * dy + dz * dz)
            return acc

        cminv[pl.ds(j * _L, _L)] = lax.fori_loop(0, _RS // _L, nstep, inf16)
        return c

    lax.fori_loop(0, _M // _L, colblk, 0)

    pltpu.sync_copy(rminv, row_o.at[box, pl.ds(r0, _RS)])
    pltpu.sync_copy(cminv, col_o.at[box, sh])


def _tc_min_body(ax_ref, ay_ref, az_ref, bx_ref, by_ref, bz_ref,
                 rowmin_ref, colminp_ref):
    axb = ax_ref[0, 0, 0][:, None]
    ayb = ay_ref[0, 0, 0][:, None]
    azb = az_ref[0, 0, 0][:, None]
    bxb = bx_ref[0, 0][None, :]
    byb = by_ref[0, 0][None, :]
    bzb = bz_ref[0, 0][None, :]
    d2 = (jnp.square(axb - bxb) + jnp.square(ayb - byb)
          + jnp.square(azb - bzb))
    rowmin_ref[0, 0, 0] = jnp.min(d2, axis=1)
    colminp_ref[0, 0, 0] = jnp.min(d2, axis=0)


_tc_min = pl.pallas_call(
    _tc_min_body,
    grid=(_B, _NRB),
    in_specs=[
        pl.BlockSpec((1, 1, 1, _RB), lambda b, r: (b, (_NSC // _RB) + r, 0, 0)),
        pl.BlockSpec((1, 1, 1, _RB), lambda b, r: (b, (_NSC // _RB) + r, 0, 0)),
        pl.BlockSpec((1, 1, 1, _RB), lambda b, r: (b, (_NSC // _RB) + r, 0, 0)),
        pl.BlockSpec((1, 1, _M), lambda b, r: (b, 0, 0)),
        pl.BlockSpec((1, 1, _M), lambda b, r: (b, 0, 0)),
        pl.BlockSpec((1, 1, _M), lambda b, r: (b, 0, 0)),
    ],
    out_specs=[
        pl.BlockSpec((1, 1, 1, _RB), lambda b, r: (b, r, 0, 0)),
        pl.BlockSpec((1, 1, 1, _M), lambda b, r: (b, r, 0, 0)),
    ],
    out_shape=[
        jax.ShapeDtypeStruct((_B, _NRB, 1, _RB), jnp.float32),
        jax.ShapeDtypeStruct((_B, _NRB, 1, _M), jnp.float32),
    ],
)


def _finish_body(rs_ref, rt_ref, cs_ref, ct_ref, out_ref):
    rt2 = rt_ref[...].reshape(_B, _NTC)
    rm2 = jnp.concatenate([rs_ref[...], rt2], axis=1)
    cm2 = jnp.minimum(jnp.min(cs_ref[...], axis=1),
                      jnp.min(ct_ref[...].reshape(_B, _NRB, _M), axis=1))
    rm = jnp.sqrt(jnp.maximum(rm2, 1e-12))
    cm = jnp.sqrt(jnp.maximum(cm2, 1e-12))
    lb = jnp.mean(cm, axis=1)
    lf = jnp.mean(rm, axis=1)
    lm = jnp.max(rm, axis=1)
    out_ref[...] = jnp.mean(5.0 * lb + lf + lm).reshape(1, 1)


_finish = pl.pallas_call(
    _finish_body,
    out_shape=jax.ShapeDtypeStruct((1, 1), jnp.float32),
)


@jax.jit
def kernel(sampled_lidar_list, surface_points):
    a = sampled_lidar_list
    b = surface_points
    ax, ay, az = (a[:, :, 0], a[:, :, 1], a[:, :, 2])
    bx, by, bz = (b[:, :, 0], b[:, :, 1], b[:, :, 2])
    rowmin_sc, colmin_sc = _sc_min(ax, ay, az, bx, by, bz)
    nrb_tot = _N // _RB
    ax4 = ax.reshape(_B, nrb_tot, 1, _RB)
    ay4 = ay.reshape(_B, nrb_tot, 1, _RB)
    az4 = az.reshape(_B, nrb_tot, 1, _RB)
    bx3 = bx.reshape(_B, 1, _M)
    by3 = by.reshape(_B, 1, _M)
    bz3 = bz.reshape(_B, 1, _M)
    rowmin_tc, colmin_tc = _tc_min(ax4, ay4, az4, bx3, by3, bz3)
    return _finish(rowmin_sc, rowmin_tc, colmin_sc, colmin_tc)[0, 0]
